# baseline (speedup 1.0000x reference)
import jax
import jax.numpy as jnp
from jax import lax
from jax.experimental import pallas as pl
from jax.experimental.pallas import tpu as pltpu
from jax.experimental.pallas import tpu_sc as plsc

VOCAB = 100000
EMB = 128
BATCH = 4096
SEQ = 50
HIDDEN = 64
OUT = 10

NUM_CORES = 2
NUM_SUBCORES = 16
NW = NUM_CORES * NUM_SUBCORES
ROWS_PER_W = BATCH // NW
LANES = 16
NCH = EMB // LANES


def _sc_body(xt_hbm, table_hbm, out_hbm, idx_v, acc, sem_x, sem):
  wid = lax.axis_index("s") * NUM_CORES + lax.axis_index("c")
  base = wid * ROWS_PER_W

  cp = pltpu.async_copy(xt_hbm.at[wid], idx_v, sem_x)

  zeros = jnp.zeros((LANES,), jnp.float32)

  def zero_body(r, _):
    for c in range(NCH):
      acc[r, pl.ds(c * LANES, LANES)] = zeros
    return 0

  lax.fori_loop(0, ROWS_PER_W, zero_body, 0)
  cp.wait()

  def fire(j, _):
    pltpu.async_copy(table_hbm.at[idx_v.at[j]], acc, sem, add=True)
    return 0

  lax.fori_loop(0, SEQ, fire, 0)

  def drain(j, _):
    pltpu.make_async_copy(table_hbm.at[idx_v.at[0]], acc, sem).wait()
    return 0

  lax.fori_loop(0, SEQ, drain, 0)

  pltpu.sync_copy(acc, out_hbm.at[pl.ds(base, ROWS_PER_W), :])


def _sc_gather_sum(xt, table):
  mesh = plsc.VectorSubcoreMesh(core_axis_name="c", subcore_axis_name="s")
  k = pl.kernel(
      _sc_body,
      out_type=jax.ShapeDtypeStruct((BATCH, EMB), jnp.float32),
      mesh=mesh,
      compiler_params=pltpu.CompilerParams(skip_device_barrier=True),
      scratch_types=[
          pltpu.VMEM((SEQ, ROWS_PER_W), jnp.int32),
          pltpu.VMEM((ROWS_PER_W, EMB), jnp.float32),
          pltpu.SemaphoreType.DMA,
          pltpu.SemaphoreType.DMA,
      ],
  )
  return k(xt, table)


def _mlp_body(sums_ref, len_ref, w1_ref, b1_ref, w2_ref, b2_ref, out_ref):
  s = sums_ref[...]
  inv = 1.0 / len_ref[...].astype(jnp.float32)
  rep = s * inv
  h = lax.dot_general(rep, w1_ref[...], (((1,), (1,)), ((), ())),
                      preferred_element_type=jnp.float32)
  h = jnp.maximum(h + b1_ref[...], 0.0)
  o = lax.dot_general(h, w2_ref[...], (((1,), (1,)), ((), ())),
                      preferred_element_type=jnp.float32)
  out_ref[...] = o + b2_ref[...]


def _tc_mlp(sums, lengths2, W1, b1, W2, b2):
  return pl.pallas_call(
      _mlp_body,
      out_shape=jax.ShapeDtypeStruct((BATCH, OUT), jnp.float32),
  )(sums, lengths2, W1, b1.reshape(1, HIDDEN), W2, b2.reshape(1, OUT))


def kernel(x, lengths, table, W1, b1, W2, b2):
  xt = x.reshape(NW, ROWS_PER_W, SEQ).transpose(0, 2, 1)
  sums = _sc_gather_sum(xt, table)
  return _tc_mlp(sums, lengths.reshape(BATCH, 1), W1, b1, W2, b2)
